# Initial kernel scaffold; baseline (speedup 1.0000x reference)
import functools

import jax
import jax.numpy as jnp
from jax import lax
from jax.experimental import pallas as pl
from jax.experimental.pallas import tpu as pltpu
from jax.experimental.pallas import tpu_sc as plsc

B, C, K, H, W = 8, 2, 1024, 512, 512
NUM_CELLS = 2 * B * C
LANES = 16
CHUNK = 128
NCHUNK = K // CHUNK
SUB = CHUNK // LANES

_mesh = plsc.VectorSubcoreMesh(core_axis_name="c", subcore_axis_name="s")


@functools.partial(
    pl.kernel,
    out_type=jax.ShapeDtypeStruct((NUM_CELLS, LANES), jnp.float32),
    mesh=_mesh,
    scratch_types=[
        pltpu.VMEM((4, K), jnp.int32),
        pltpu.VMEM((NCHUNK, CHUNK), jnp.int32),
        pltpu.VMEM((NCHUNK, CHUNK), jnp.int32),
        pltpu.VMEM((NCHUNK, CHUNK), jnp.float32),
        pltpu.VMEM((NCHUNK, CHUNK), jnp.float32),
        pltpu.VMEM((LANES,), jnp.float32),
        pltpu.SemaphoreType.DMA,
    ],
)
def _bd_loss_sc(pred_hbm, ints_hbm, out_hbm,
                ints_v, bidx_v, didx_v, bvals_v, dvals_v, acc_v, sem):
    cell = lax.axis_index("s") * 2 + lax.axis_index("c")
    bc = lax.rem(cell, 16)
    base = bc * (H * W)
    good_tile = jnp.logical_or(cell < 16, lax.rem(cell, 2) == 1)

    pltpu.sync_copy(ints_hbm.at[cell], ints_v)

    for j in range(NCHUNK):
        for t in range(SUB):
            o = j * CHUNK + t * LANES
            bx = ints_v[0, pl.ds(o, LANES)]
            by = ints_v[1, pl.ds(o, LANES)]
            dx = ints_v[2, pl.ds(o, LANES)]
            dy = ints_v[3, pl.ds(o, LANES)]
            bidx_v[j, pl.ds(t * LANES, LANES)] = base + bx * W + by
            didx_v[j, pl.ds(t * LANES, LANES)] = base + dx * W + dy

    copies = []
    for j in range(NCHUNK):
        copies.append(pltpu.make_async_copy(
            pred_hbm.at[bidx_v.at[j]], bvals_v.at[j], sem))
        copies.append(pltpu.make_async_copy(
            pred_hbm.at[didx_v.at[j]], dvals_v.at[j], sem))
    for cp in copies:
        cp.start()
    for cp in copies:
        cp.wait()

    lane = lax.iota(jnp.int32, LANES)
    acc = jnp.zeros((LANES,), jnp.float32)
    for j in range(NCHUNK):
        for t in range(SUB):
            s = pl.ds(t * LANES, LANES)
            d = bvals_v[j, s] - dvals_v[j, s]
            d2 = d * d
            if j == 0 and t == 0:
                flip = jnp.logical_and(lane == 0, good_tile)
                d2 = jnp.where(flip, 1.0 - d2, d2)
            acc = acc + d2

    acc_v[...] = acc
    pltpu.sync_copy(acc_v, out_hbm.at[cell])


def kernel(prediction, intervals_comp_0, intervals_comp_1):
    ints = jnp.stack([intervals_comp_0, intervals_comp_1])
    ints = ints.transpose(0, 1, 2, 4, 5, 3).reshape(NUM_CELLS, 4, K)
    partials = _bd_loss_sc(prediction.reshape(-1), ints)
    return jnp.sum(partials)


# speedup vs baseline: 1.0927x; 1.0927x over previous
import functools

import jax
import jax.numpy as jnp
from jax import lax
from jax.experimental import pallas as pl
from jax.experimental.pallas import tpu as pltpu
from jax.experimental.pallas import tpu_sc as plsc

B, C, K, H, W = 8, 2, 1024, 512, 512
NUM_CELLS = 2 * B * C
LANES = 16
CHUNK = 128
NCHUNK = K // CHUNK
SUB = CHUNK // LANES

_mesh = plsc.VectorSubcoreMesh(core_axis_name="c", subcore_axis_name="s")


@functools.partial(
    pl.kernel,
    out_type=jax.ShapeDtypeStruct((NUM_CELLS, LANES), jnp.float32),
    mesh=_mesh,
    scratch_types=[
        pltpu.VMEM((4, K), jnp.int32),
        pltpu.VMEM((NCHUNK, CHUNK), jnp.int32),
        pltpu.VMEM((NCHUNK, CHUNK), jnp.int32),
        pltpu.VMEM((NCHUNK, CHUNK), jnp.float32),
        pltpu.VMEM((NCHUNK, CHUNK), jnp.float32),
        pltpu.VMEM((LANES,), jnp.float32),
        pltpu.SemaphoreType.DMA,
    ],
)
def _bd_loss_sc(pred_hbm, ints_hbm, out_hbm,
                ints_v, bidx_v, didx_v, bvals_v, dvals_v, acc_v, sem):
    cell = lax.axis_index("s") * 2 + lax.axis_index("c")
    bc = lax.rem(cell, 16)
    base = bc * (H * W)
    comp = lax.div(cell, 16)
    cls = lax.rem(cell, 2)
    good_i = lax.max(1 - comp, cls)

    pltpu.sync_copy(ints_hbm.at[cell], ints_v)

    for j in range(NCHUNK):
        for t in range(SUB):
            o = j * CHUNK + t * LANES
            bx = ints_v[0, pl.ds(o, LANES)]
            by = ints_v[1, pl.ds(o, LANES)]
            dx = ints_v[2, pl.ds(o, LANES)]
            dy = ints_v[3, pl.ds(o, LANES)]
            bidx_v[j, pl.ds(t * LANES, LANES)] = base + bx * W + by
            didx_v[j, pl.ds(t * LANES, LANES)] = base + dx * W + dy

    copies = []
    for j in range(NCHUNK):
        copies.append(pltpu.make_async_copy(
            pred_hbm.at[bidx_v.at[j]], bvals_v.at[j], sem))
        copies.append(pltpu.make_async_copy(
            pred_hbm.at[didx_v.at[j]], dvals_v.at[j], sem))
    for cp in copies:
        cp.start()
    for cp in copies:
        cp.wait()

    lane = lax.iota(jnp.int32, LANES)
    onehot0 = jnp.maximum(1 - lane, 0).astype(jnp.float32)
    flip = onehot0 * good_i.astype(jnp.float32)
    acc = jnp.zeros((LANES,), jnp.float32)
    for j in range(NCHUNK):
        for t in range(SUB):
            s = pl.ds(t * LANES, LANES)
            d = bvals_v[j, s] - dvals_v[j, s]
            d2 = d * d
            if j == 0 and t == 0:
                d2 = d2 + flip * (1.0 - 2.0 * d2)
            acc = acc + d2

    acc_v[...] = acc
    pltpu.sync_copy(acc_v, out_hbm.at[cell])


def kernel(prediction, intervals_comp_0, intervals_comp_1):
    ints = jnp.stack([intervals_comp_0, intervals_comp_1])
    ints = ints.transpose(0, 1, 2, 4, 5, 3).reshape(NUM_CELLS, 4, K)
    partials = _bd_loss_sc(prediction.reshape(-1), ints)
    return jnp.sum(partials)
